# baseline (speedup 1.0000x reference)
import jax
import jax.numpy as jnp
from jax.experimental import pallas as pl
from jax.experimental.pallas import tpu as pltpu
from jax.experimental.pallas import tpu_sc as plsc

_NC = 2
_NS = 16
_NW = _NC * _NS
_CHUNK = 24
_NBUF = 8
_BB = 64


def _sc_gather(x_3d, weight):
    _, chunks, _ = x_3d.shape
    n = x_3d.size
    _, dim = weight.shape
    toks_w = n // _NW

    mesh = plsc.VectorSubcoreMesh(core_axis_name="c", subcore_axis_name="s")

    @pl.kernel(
        out_type=jax.ShapeDtypeStruct((n, dim), weight.dtype),
        mesh=mesh,
        scratch_types=(
            [pltpu.VMEM((chunks, _CHUNK), jnp.int32)]
            + [pltpu.VMEM((_CHUNK, dim), jnp.float32) for _ in range(_NBUF)]
            + [pltpu.SemaphoreType.DMA for _ in range(2 * _NBUF)]
        ),
    )
    def gather_kernel(w_hbm, i_hbm, o_hbm, idx_v, *bufs_and_sems):
        bufs = bufs_and_sems[:_NBUF]
        gsem = bufs_and_sems[_NBUF:2 * _NBUF]
        wsem = bufs_and_sems[2 * _NBUF:]

        wid = jax.lax.axis_index("s") * _NC + jax.lax.axis_index("c")
        tok0 = wid * toks_w

        pltpu.sync_copy(i_hbm.at[wid], idx_v)

        def start_gather(c, b):
            pltpu.async_copy(w_hbm.at[idx_v.at[c]], bufs[b], gsem[b])

        def wait_gather(b):
            pltpu.make_async_copy(
                w_hbm.at[idx_v.at[0]], bufs[b], gsem[b]).wait()

        def start_write(c, b):
            pltpu.async_copy(
                bufs[b], o_hbm.at[pl.ds(tok0 + c * _CHUNK, _CHUNK)], wsem[b])

        def wait_write(b):
            pltpu.make_async_copy(
                bufs[b], o_hbm.at[pl.ds(tok0, _CHUNK)], wsem[b]).wait()

        for b in range(_NBUF):
            start_gather(b, b)

        @pl.loop(0, chunks, step=_NBUF)
        def _(c0):
            for b in range(_NBUF):
                c = c0 + b
                wait_gather(b)
                start_write(c, b)
                nxt = c + _NBUF

                @pl.when(nxt < chunks)
                def _():
                    wait_write(b)
                    start_gather(nxt, b)

        for b in range(_NBUF):
            wait_write(b)

    return gather_kernel(weight, x_3d)


def _tc_tail(dense3, batch, seq, dim):
    seqp = dense3.shape[1]

    def body(i_ref, o_ref):
        o_ref[...] = i_ref[:, :seq, :]

    return pl.pallas_call(
        body,
        grid=(batch // _BB,),
        in_specs=[pl.BlockSpec((_BB, seqp, dim), lambda i: (i, 0, 0))],
        out_specs=pl.BlockSpec((_BB, seq, dim), lambda i: (i, 0, 0)),
        out_shape=jax.ShapeDtypeStruct((batch, seq, dim), dense3.dtype),
        compiler_params=pltpu.CompilerParams(
            dimension_semantics=("parallel",)),
    )(dense3)


def kernel(x, weight):
    batch, seq = x.shape
    _, dim = weight.shape
    seqp = 24
    vocab = weight.shape[0]
    pad_idx = (
        jnp.arange(batch, dtype=x.dtype)[:, None] * (seqp - seq)
        + jnp.arange(seqp - seq, dtype=x.dtype)
    ) % vocab
    xp = jnp.concatenate([x, pad_idx], axis=1)
    n = batch * seqp
    toks_w = n // _NW
    dense = _sc_gather(xp.reshape(_NW, toks_w // _CHUNK, _CHUNK), weight)
    return _tc_tail(dense.reshape(batch, seqp, dim), batch, seq, dim)


# speedup vs baseline: 1.5248x; 1.5248x over previous
import jax
import jax.numpy as jnp
from jax.experimental import pallas as pl
from jax.experimental.pallas import tpu as pltpu
from jax.experimental.pallas import tpu_sc as plsc

_NC = 2
_NS = 16
_NW = _NC * _NS
_CHUNK = 32
_NBUF = 6
_BB = 64


def _sc_gather(x_3d, weight):
    _, chunks, _ = x_3d.shape
    n = x_3d.size
    _, dim = weight.shape
    toks_w = n // _NW

    mesh = plsc.VectorSubcoreMesh(core_axis_name="c", subcore_axis_name="s")

    @pl.kernel(
        out_type=jax.ShapeDtypeStruct((n, dim), weight.dtype),
        mesh=mesh,
        scratch_types=(
            [pltpu.VMEM((chunks, _CHUNK), jnp.int32)]
            + [pltpu.VMEM((_CHUNK, dim), jnp.float32) for _ in range(_NBUF)]
            + [pltpu.SemaphoreType.DMA for _ in range(2 * _NBUF)]
        ),
    )
    def gather_kernel(w_hbm, i_hbm, o_hbm, idx_v, *bufs_and_sems):
        bufs = bufs_and_sems[:_NBUF]
        gsem = bufs_and_sems[_NBUF:2 * _NBUF]
        wsem = bufs_and_sems[2 * _NBUF:]

        wid = jax.lax.axis_index("s") * _NC + jax.lax.axis_index("c")
        tok0 = wid * toks_w

        pltpu.sync_copy(i_hbm.at[wid], idx_v)

        def start_gather(c, b):
            pltpu.async_copy(w_hbm.at[idx_v.at[c]], bufs[b], gsem[b])

        def wait_gather(b):
            pltpu.make_async_copy(
                w_hbm.at[idx_v.at[0]], bufs[b], gsem[b]).wait()

        def start_write(c, b):
            pltpu.async_copy(
                bufs[b], o_hbm.at[pl.ds(tok0 + c * _CHUNK, _CHUNK)], wsem[b])

        def wait_write(b):
            pltpu.make_async_copy(
                bufs[b], o_hbm.at[pl.ds(tok0, _CHUNK)], wsem[b]).wait()

        for b in range(_NBUF):
            start_gather(b, b)

        @pl.loop(0, chunks, step=_NBUF)
        def _(c0):
            for b in range(_NBUF):
                c = c0 + b
                wait_gather(b)
                start_write(c, b)
                nxt = c + _NBUF

                @pl.when(nxt < chunks)
                def _():
                    wait_write(b)
                    start_gather(nxt, b)

        for b in range(_NBUF):
            wait_write(b)

    return gather_kernel(weight, x_3d)


def _tc_tail(dense3, batch, seq, dim):
    seqp = dense3.shape[1]

    def body(i_ref, o_ref):
        o_ref[...] = i_ref[:, :seq, :]

    return pl.pallas_call(
        body,
        grid=(batch // _BB,),
        in_specs=[pl.BlockSpec((_BB, seqp, dim), lambda i: (i, 0, 0))],
        out_specs=pl.BlockSpec((_BB, seq, dim), lambda i: (i, 0, 0)),
        out_shape=jax.ShapeDtypeStruct((batch, seq, dim), dense3.dtype),
        compiler_params=pltpu.CompilerParams(
            dimension_semantics=("parallel",)),
    )(dense3)


def kernel(x, weight):
    batch, seq = x.shape
    _, dim = weight.shape
    seqp = 24
    vocab = weight.shape[0]
    pad_idx = (
        jnp.arange(batch, dtype=x.dtype)[:, None] * (seqp - seq)
        + jnp.arange(seqp - seq, dtype=x.dtype)
    ) % vocab
    xp = jnp.concatenate([x, pad_idx], axis=1)
    n = batch * seqp
    toks_w = n // _NW
    dense = _sc_gather(xp.reshape(_NW, toks_w // _CHUNK, _CHUNK), weight)
    return dense.reshape(batch, seqp, dim)[:, :seq, :]


# speedup vs baseline: 1.5324x; 1.0050x over previous
import jax
import jax.numpy as jnp
from jax.experimental import pallas as pl
from jax.experimental.pallas import tpu as pltpu
from jax.experimental.pallas import tpu_sc as plsc

_NC = 2
_NS = 16
_NW = _NC * _NS
_CHUNK = 24
_NBUF = 8


def _sc_gather(x_3d, weight):
    _, chunks, _ = x_3d.shape
    n = x_3d.size
    _, dim = weight.shape
    toks_w = n // _NW

    mesh = plsc.VectorSubcoreMesh(core_axis_name="c", subcore_axis_name="s")

    @pl.kernel(
        out_type=jax.ShapeDtypeStruct((n, dim), weight.dtype),
        mesh=mesh,
        scratch_types=(
            [pltpu.VMEM((chunks, _CHUNK), jnp.int32)]
            + [pltpu.VMEM((_CHUNK, dim), jnp.float32) for _ in range(_NBUF)]
            + [pltpu.SemaphoreType.DMA for _ in range(2 * _NBUF)]
        ),
    )
    def gather_kernel(w_hbm, i_hbm, o_hbm, idx_v, *bufs_and_sems):
        bufs = bufs_and_sems[:_NBUF]
        gsem = bufs_and_sems[_NBUF:2 * _NBUF]
        wsem = bufs_and_sems[2 * _NBUF:]

        wid = jax.lax.axis_index("s") * _NC + jax.lax.axis_index("c")
        tok0 = wid * toks_w

        pltpu.sync_copy(i_hbm.at[wid], idx_v)

        def start_gather(c, b):
            pltpu.async_copy(w_hbm.at[idx_v.at[c]], bufs[b], gsem[b])

        def wait_gather(b):
            pltpu.make_async_copy(
                w_hbm.at[idx_v.at[0]], bufs[b], gsem[b]).wait()

        def start_write(c, b):
            pltpu.async_copy(
                bufs[b], o_hbm.at[pl.ds(tok0 + c * _CHUNK, _CHUNK)], wsem[b])

        def wait_write(b):
            pltpu.make_async_copy(
                bufs[b], o_hbm.at[pl.ds(tok0, _CHUNK)], wsem[b]).wait()

        for b in range(_NBUF):
            start_gather(b, b)

        @pl.loop(0, chunks, step=_NBUF)
        def _(c0):
            for b in range(_NBUF):
                c = c0 + b
                wait_gather(b)
                start_write(c, b)
                nxt = c + _NBUF

                @pl.when(nxt < chunks)
                def _():
                    wait_write(b)
                    start_gather(nxt, b)

        for b in range(_NBUF):
            wait_write(b)

    return gather_kernel(weight, x_3d)


def kernel(x, weight):
    batch, seq = x.shape
    _, dim = weight.shape
    seqp = 24
    vocab = weight.shape[0]
    pad_idx = (
        jnp.arange(batch, dtype=x.dtype)[:, None] * (seqp - seq)
        + jnp.arange(seqp - seq, dtype=x.dtype)
    ) % vocab
    xp = jnp.concatenate([x, pad_idx], axis=1)
    n = batch * seqp
    toks_w = n // _NW
    dense = _sc_gather(xp.reshape(_NW, toks_w // _CHUNK, _CHUNK), weight)
    return dense.reshape(batch, seqp, dim)[:, :seq, :]
